# baseline (speedup 1.0000x reference)
import functools

import jax
import jax.numpy as jnp
from jax.experimental import pallas as pl

HIDDEN = 2048
EXPERTS = 64
K = 8
BT = 2048
NSPLIT = 2
HC = HIDDEN // NSPLIT


def _router_kernel(*refs):
    h_refs = refs[:NSPLIT]
    w_refs = refs[NSPLIT:2 * NSPLIT]
    b_ref = refs[2 * NSPLIT]
    logits_ref, idx_ref, wts_ref = refs[2 * NSPLIT + 1:]

    b = b_ref[...]
    dn = (((1,), (1,)), ((), ()))
    logits = jnp.zeros((BT, EXPERTS), jnp.float32)
    lt = jnp.zeros((EXPERTS, BT), jnp.float32)
    for h_ref, w_ref in zip(h_refs, w_refs):
        h = h_ref[...]
        w = w_ref[...]
        logits = logits + jax.lax.dot_general(
            h, w, dn, preferred_element_type=jnp.float32)
        lt = lt + jax.lax.dot_general(
            w, h, dn, preferred_element_type=jnp.float32)
    logits = logits + b[None, :]
    lt = lt + b[:, None]
    logits_ref[...] = logits

    iota = jax.lax.broadcasted_iota(jnp.int32, (EXPERTS, BT), 0)
    work = lt
    vals = []
    idxs = []
    for _ in range(K):
        m = jnp.max(work, axis=0, keepdims=True)
        is_max = work == m
        idx = jnp.min(jnp.where(is_max, iota, EXPERTS), axis=0, keepdims=True)
        vals.append(m)
        idxs.append(idx)
        work = jnp.where(iota == idx, -jnp.inf, work)
    top_v = jnp.concatenate(vals, axis=0)
    top_i = jnp.concatenate(idxs, axis=0)

    e = jnp.exp(top_v - top_v[:1])
    wts = e / jnp.sum(e, axis=0, keepdims=True)

    idx_ref[...] = top_i.T
    wts_ref[...] = wts.T


@functools.partial(jax.jit, static_argnames=())
def kernel(hidden, W, b):
    n_tokens = hidden.shape[0]
    grid = (n_tokens // BT,)
    h_specs = [
        pl.BlockSpec((BT, HC), lambda i, c=c: (i, c)) for c in range(NSPLIT)
    ]
    w_specs = [
        pl.BlockSpec((EXPERTS, HC), lambda i, c=c: (0, c)) for c in range(NSPLIT)
    ]
    logits, idx, wts = pl.pallas_call(
        _router_kernel,
        grid=grid,
        in_specs=h_specs + w_specs + [pl.BlockSpec((EXPERTS,), lambda i: (0,))],
        out_specs=[
            pl.BlockSpec((BT, EXPERTS), lambda i: (i, 0)),
            pl.BlockSpec((BT, K), lambda i: (i, 0)),
            pl.BlockSpec((BT, K), lambda i: (i, 0)),
        ],
        out_shape=[
            jax.ShapeDtypeStruct((n_tokens, EXPERTS), jnp.float32),
            jax.ShapeDtypeStruct((n_tokens, K), jnp.int32),
            jax.ShapeDtypeStruct((n_tokens, K), jnp.float32),
        ],
    )(*([hidden] * NSPLIT + [W] * NSPLIT + [b]))
    return idx, wts, logits


# speedup vs baseline: 1.0821x; 1.0821x over previous
import functools

import jax
import jax.numpy as jnp
from jax.experimental import pallas as pl

HIDDEN = 2048
EXPERTS = 64
K = 8
BT = 2048
SB = 512
NSB = BT // SB


def _topk_softmax(lt):
    iota = jax.lax.broadcasted_iota(jnp.int32, (EXPERTS, SB), 0)
    work = lt
    vals = []
    idxs = []
    for _ in range(K):
        m = jnp.max(work, axis=0, keepdims=True)
        is_max = work == m
        idx = jnp.min(jnp.where(is_max, iota, EXPERTS), axis=0, keepdims=True)
        vals.append(m)
        idxs.append(idx)
        work = jnp.where(iota == idx, -jnp.inf, work)
    top_v = jnp.concatenate(vals, axis=0)
    top_i = jnp.concatenate(idxs, axis=0)
    e = jnp.exp(top_v - top_v[:1])
    wts = e / jnp.sum(e, axis=0, keepdims=True)
    return top_i, wts


def _router_kernel(h_ref, w_ref, b_ref, logits_ref, idx_ref, wts_ref):
    w = w_ref[...]
    b = b_ref[...]
    dn = (((1,), (1,)), ((), ()))
    for s in range(NSB):
        h = h_ref[pl.ds(s * SB, SB), :]
        lt = jax.lax.dot_general(
            w, h, dn, preferred_element_type=jnp.float32,
        ) + b[:, None]
        logits_ref[pl.ds(s * SB, SB), :] = lt.T
        top_i, wts = _topk_softmax(lt)
        idx_ref[pl.ds(s * SB, SB), :] = top_i.T
        wts_ref[pl.ds(s * SB, SB), :] = wts.T


@functools.partial(jax.jit, static_argnames=())
def kernel(hidden, W, b):
    n_tokens = hidden.shape[0]
    grid = (n_tokens // BT,)
    logits, idx, wts = pl.pallas_call(
        _router_kernel,
        grid=grid,
        in_specs=[
            pl.BlockSpec((BT, HIDDEN), lambda i: (i, 0)),
            pl.BlockSpec((EXPERTS, HIDDEN), lambda i: (0, 0)),
            pl.BlockSpec((EXPERTS,), lambda i: (0,)),
        ],
        out_specs=[
            pl.BlockSpec((BT, EXPERTS), lambda i: (i, 0)),
            pl.BlockSpec((BT, K), lambda i: (i, 0)),
            pl.BlockSpec((BT, K), lambda i: (i, 0)),
        ],
        out_shape=[
            jax.ShapeDtypeStruct((n_tokens, EXPERTS), jnp.float32),
            jax.ShapeDtypeStruct((n_tokens, K), jnp.int32),
            jax.ShapeDtypeStruct((n_tokens, K), jnp.float32),
        ],
    )(hidden, W, b)
    return idx, wts, logits
